# speedup vs baseline: 6.3361x; 6.3361x over previous
import functools

import jax
import jax.numpy as jnp
from jax.experimental import pallas as pl
from jax.experimental.pallas import tpu as pltpu

_N_BINS = 10
_BB = 32


def _qice_kernel(pred_ref, truth_ref, out_ref):
    x = pred_ref[...]
    tv = truth_ref[...]
    t = tv[:, :, None]

    mask = x < t
    r = jnp.sum(mask.astype(jnp.float32), axis=-1).astype(jnp.int32)
    a = jnp.max(jnp.where(mask, x, -jnp.inf), axis=-1)
    b = jnp.min(jnp.where(mask, jnp.inf, x), axis=-1)

    base = jnp.where(r >= 1, 1 + jnp.minimum((r - 1) // 10, 9), 0)
    base = base + jnp.where(r == 100, 1, 0)

    amb = (r % 10 == 0) & (r >= 10) & (r <= 90)
    kf = (r // 10).astype(jnp.float32)
    qv = kf * jnp.float32(0.1)
    idx = qv * jnp.float32(99.0)
    hw = idx - jnp.floor(idx)
    lw = jnp.float32(1.0) - hw
    interp = a * lw + b * hw
    m = base + jnp.where(amb & (interp < tv), 1, 0)

    bin0 = jnp.clip(m, 1, _N_BINS) - 1

    one_hot = (bin0[:, :, None] == jax.lax.broadcasted_iota(
        jnp.int32, (1, 1, _N_BINS), 2)).astype(jnp.float32)
    hist = jnp.sum(one_hot, axis=(0, 1))

    @pl.when(pl.program_id(0) == 0)
    def _init():
        out_ref[...] = jnp.zeros_like(out_ref)

    out_ref[0, :] += hist


@jax.jit
def kernel(prediction, truth):
    nb = prediction.shape[0]
    grid = (nb // _BB,)
    out = pl.pallas_call(
        _qice_kernel,
        grid=grid,
        in_specs=[
            pl.BlockSpec((_BB, 256, 100), lambda i: (i, 0, 0)),
            pl.BlockSpec((_BB, 256), lambda i: (i, 0)),
        ],
        out_specs=pl.BlockSpec((1, _N_BINS), lambda i: (0, 0)),
        out_shape=jax.ShapeDtypeStruct((1, _N_BINS), jnp.float32),
    )(prediction, truth)
    return out[0]


# speedup vs baseline: 17.0475x; 2.6905x over previous
import functools

import jax
import jax.numpy as jnp
from jax.experimental import pallas as pl
from jax.experimental.pallas import tpu as pltpu

_N_BINS = 10
_BB = 32


def _qice_kernel(pred_ref, truth_ref, out_ref):
    x = jnp.swapaxes(pred_ref[...], 1, 2)
    tv = truth_ref[...]
    t = tv[:, None, :]

    mask = x < t
    r = jnp.sum(mask.astype(jnp.float32), axis=1).astype(jnp.int32)
    a = jnp.max(jnp.where(mask, x, -jnp.inf), axis=1)
    b = jnp.min(jnp.where(mask, jnp.inf, x), axis=1)

    base = jnp.where(r >= 1, 1 + jnp.minimum((r - 1) // 10, 9), 0)
    base = base + jnp.where(r == 100, 1, 0)

    amb = (r % 10 == 0) & (r >= 10) & (r <= 90)
    kf = (r // 10).astype(jnp.float32)
    qv = kf * jnp.float32(0.1)
    idx = qv * jnp.float32(99.0)
    hw = idx - jnp.floor(idx)
    lw = jnp.float32(1.0) - hw
    interp = a * lw + b * hw
    m = base + jnp.where(amb & (interp < tv), 1, 0)

    bin0 = jnp.clip(m, 1, _N_BINS) - 1

    one_hot = (bin0[:, :, None] == jax.lax.broadcasted_iota(
        jnp.int32, (1, 1, _N_BINS), 2)).astype(jnp.float32)
    hist = jnp.sum(one_hot, axis=(0, 1))

    @pl.when(pl.program_id(0) == 0)
    def _init():
        out_ref[...] = jnp.zeros_like(out_ref)

    out_ref[0, :] += hist


@jax.jit
def kernel(prediction, truth):
    nb = prediction.shape[0]
    grid = (nb // _BB,)
    out = pl.pallas_call(
        _qice_kernel,
        grid=grid,
        in_specs=[
            pl.BlockSpec((_BB, 256, 100), lambda i: (i, 0, 0)),
            pl.BlockSpec((_BB, 256), lambda i: (i, 0)),
        ],
        out_specs=pl.BlockSpec((1, _N_BINS), lambda i: (0, 0)),
        out_shape=jax.ShapeDtypeStruct((1, _N_BINS), jnp.float32),
    )(prediction, truth)
    return out[0]
